# baseline (speedup 1.0000x reference)
import functools
import math

import jax
import jax.numpy as jnp
from jax import lax
from jax.experimental import pallas as pl
from jax.experimental.pallas import tpu as pltpu
from jax.experimental.pallas import tpu_sc as plsc

VOCAB = 1000000
EMB_DIM = 64
BATCH = 16384
HIST = 50
NC = 2
NS = 16
NW = NC * NS
BW = BATCH // NW
BPW = HIST * BW
G = 128
NJ = BW // G
NBLK = HIST * NJ
NB = 3
LANES = 16
SCALE = math.sqrt(EMB_DIM)
_MESH = plsc.VectorSubcoreMesh(core_axis_name="c", subcore_axis_name="s")


def _worker_id():
    return lax.axis_index("s") * NC + lax.axis_index("c")


def _stage_idx_body(xt_hbm, out_hbm, tbuf, osem):
    w = _worker_id()
    for jj in range(NJ):
        j_col = w * NJ + jj
        for i in range((HIST + 7) // 8):
            rows = min(8, HIST - 8 * i)
            buf = tbuf[(i + jj) % 2]
            pltpu.sync_copy(
                xt_hbm.at[pl.ds(8 * i, rows), pl.ds(128 * j_col, 128)],
                buf.at[pl.ds(0, rows)],
            )
            for h8 in range(rows):
                h = 8 * i + h8
                pltpu.async_copy(
                    buf.at[h8],
                    out_hbm.at[pl.ds(w * BPW + h * BW + jj * G, G)],
                    osem,
                )
            for h8 in range(rows):
                pltpu.make_async_copy(
                    buf.at[h8], out_hbm.at[pl.ds(0, G)], osem
                ).wait()


@functools.partial(
    pl.kernel,
    mesh=_MESH,
    out_type=jax.ShapeDtypeStruct((BATCH * HIST,), jnp.int32),
    scratch_types=[
        [pltpu.VMEM((8, 128), jnp.int32) for _ in range(2)],
        pltpu.SemaphoreType.DMA,
    ],
    compiler_params=pltpu.CompilerParams(
        use_tc_tiling_on_sc=True, needs_layout_passes=False
    ),
)
def _stage_idx_sc(xt_hbm, out_hbm, tbuf, osem):
    _stage_idx_body(xt_hbm, out_hbm, tbuf, osem)


def _embed_body(x1d_hbm, table_hbm, out_hbm, idx_all, gbuf, stage, gsem, osem):
    wid = _worker_id()
    b0 = wid * BW

    pltpu.sync_copy(x1d_hbm.at[pl.ds(wid * BPW, BPW)], idx_all)

    def fire_gather(g, b):
        pltpu.async_copy(
            table_hbm.at[idx_all.at[pl.ds(g * G, G)]], gbuf[b], gsem[b]
        )

    def drain_gather(b):
        pltpu.make_async_copy(
            table_hbm.at[idx_all.at[pl.ds(0, G)]], gbuf[b], gsem[b]
        ).wait()

    def fire_store(g, b):
        h = g // NJ
        jj = g % NJ
        pltpu.async_copy(
            stage[b], out_hbm.at[h, :, pl.ds(b0 + jj * G, G)], osem[b]
        )

    def drain_store(b):
        pltpu.make_async_copy(
            stage[b], out_hbm.at[0, :, pl.ds(0, G)], osem[b]
        ).wait()

    iota16 = lax.broadcasted_iota(jnp.int32, (LANES,), 0)

    def transpose_scale(b):
        @plsc.parallel_loop(0, G // LANES, unroll=2)
        def _(m):
            rows16 = m * LANES + iota16
            for e0 in range(0, EMB_DIM, 16):
                vals = [
                    plsc.load_gather(
                        gbuf[b],
                        [rows16, jnp.full((LANES,), e0 + k, jnp.int32)],
                    )
                    * SCALE
                    for k in range(16)
                ]
                for k in range(16):
                    stage[b][e0 + k, pl.ds(m * LANES, LANES)] = vals[k]

    def halfstep(g, b, fire_next):
        bn = (b + 1) % NB

        @pl.when(g >= 2)
        def _():
            drain_store(bn)

        if fire_next:
            fire_gather(g + 1, bn)
        drain_gather(b)
        transpose_scale(b)
        fire_store(g, b)

    fire_gather(0, 0)

    nsup = (NBLK - 2) // NB

    def superstep(t, c2):
        for k in range(NB):
            halfstep(t * NB + k, k, fire_next=True)
        return c2

    lax.fori_loop(0, nsup, superstep, 0)

    halfstep(NBLK - 2, 0, fire_next=True)
    halfstep(NBLK - 1, 1, fire_next=False)
    drain_store(0)
    drain_store(1)


@functools.partial(
    pl.kernel,
    mesh=_MESH,
    out_type=jax.ShapeDtypeStruct((HIST, EMB_DIM, BATCH), jnp.float32),
    scratch_types=[
        pltpu.VMEM((BPW,), jnp.int32),
        [pltpu.VMEM((G, EMB_DIM), jnp.float32) for _ in range(NB)],
        [pltpu.VMEM((EMB_DIM, G), jnp.float32) for _ in range(NB)],
        [pltpu.SemaphoreType.DMA for _ in range(NB)],
        [pltpu.SemaphoreType.DMA for _ in range(NB)],
    ],
    compiler_params=pltpu.CompilerParams(
        use_tc_tiling_on_sc=False, needs_layout_passes=False
    ),
)
def _embed_sc(x1d_hbm, table_hbm, out_hbm, idx_all, gbuf, stage, gsem, osem):
    _embed_body(x1d_hbm, table_hbm, out_hbm, idx_all, gbuf, stage, gsem, osem)


def kernel(x, table):
    xt = x.T.astype(jnp.int32)
    x1d = _stage_idx_sc(xt)
    outp = _embed_sc(x1d, table)
    return outp.transpose(2, 0, 1)


# speedup vs baseline: 1.0022x; 1.0022x over previous
import functools
import math

import jax
import jax.numpy as jnp
from jax import lax
from jax.experimental import pallas as pl
from jax.experimental.pallas import tpu as pltpu
from jax.experimental.pallas import tpu_sc as plsc

VOCAB = 1000000
EMB_DIM = 64
BATCH = 16384
HIST = 50
NC = 2
NS = 16
NW = NC * NS
BW = BATCH // NW
BPW = HIST * BW
G = 128
NJ = BW // G
NBLK = HIST * NJ
NB = 3
LANES = 16
SCALE = math.sqrt(EMB_DIM)
_MESH = plsc.VectorSubcoreMesh(core_axis_name="c", subcore_axis_name="s")


def _worker_id():
    return lax.axis_index("s") * NC + lax.axis_index("c")


def _stage_idx_body(xt_hbm, out_hbm, tbuf, osem):
    w = _worker_id()
    for jj in range(NJ):
        j_col = w * NJ + jj
        for i in range((HIST + 7) // 8):
            rows = min(8, HIST - 8 * i)
            buf = tbuf[(i + jj) % 2]
            pltpu.sync_copy(
                xt_hbm.at[pl.ds(8 * i, rows), pl.ds(128 * j_col, 128)],
                buf.at[pl.ds(0, rows)],
            )
            for h8 in range(rows):
                h = 8 * i + h8
                pltpu.async_copy(
                    buf.at[h8],
                    out_hbm.at[pl.ds(w * BPW + h * BW + jj * G, G)],
                    osem,
                )
            for h8 in range(rows):
                pltpu.make_async_copy(
                    buf.at[h8], out_hbm.at[pl.ds(0, G)], osem
                ).wait()


@functools.partial(
    pl.kernel,
    mesh=_MESH,
    out_type=jax.ShapeDtypeStruct((BATCH * HIST,), jnp.int32),
    scratch_types=[
        [pltpu.VMEM((8, 128), jnp.int32) for _ in range(2)],
        pltpu.SemaphoreType.DMA,
    ],
    compiler_params=pltpu.CompilerParams(
        use_tc_tiling_on_sc=True, needs_layout_passes=False
    ),
)
def _stage_idx_sc(xt_hbm, out_hbm, tbuf, osem):
    _stage_idx_body(xt_hbm, out_hbm, tbuf, osem)


NVB = VOCAB // 128
_VB_BASE = NVB // NW
_VB_EXTRA = NVB % NW


def _detile_body(tabt_hbm, t2_hbm, tbuf, sbuf, isem, osem):
    w = _worker_id()
    nj = jnp.where(w < _VB_EXTRA, _VB_BASE + 1, _VB_BASE)
    j0 = w * _VB_BASE + jnp.minimum(w, _VB_EXTRA)
    iota16 = lax.broadcasted_iota(jnp.int32, (LANES,), 0)

    def fire_in(j, b):
        for i in range(8):
            pltpu.async_copy(
                tabt_hbm.at[pl.ds(8 * i, 8), pl.ds(128 * j, 128)],
                tbuf[b].at[i],
                isem[b],
            )

    def drain_in(b):
        for i in range(8):
            pltpu.make_async_copy(
                tabt_hbm.at[pl.ds(0, 8), pl.ds(0, 128)], tbuf[b].at[i], isem[b]
            ).wait()

    def fire_out(j, b):
        pltpu.async_copy(
            sbuf[b], t2_hbm.at[pl.ds(j * 128 * EMB_DIM, 128 * EMB_DIM)], osem[b]
        )

    def drain_out(b):
        pltpu.make_async_copy(
            sbuf[b], t2_hbm.at[pl.ds(0, 128 * EMB_DIM)], osem[b]
        ).wait()

    idx_c = [
        jnp.asarray(
            [((e0 + k) // 8) for k in range(LANES)], dtype=jnp.int32
        )
        for e0 in range(0, EMB_DIM, LANES)
    ]
    idx_s = [
        jnp.asarray(
            [((e0 + k) % 8) for k in range(LANES)], dtype=jnp.int32
        )
        for e0 in range(0, EMB_DIM, LANES)
    ]

    def transpose_block(b):
        @plsc.parallel_loop(0, 128, unroll=2)
        def _(v):
            vsplat = jnp.full((LANES,), 0, jnp.int32) + v
            vals = [
                plsc.load_gather(tbuf[b], [idx_c[c], idx_s[c], vsplat]) * SCALE
                for c in range(EMB_DIM // LANES)
            ]
            for c in range(EMB_DIM // LANES):
                sbuf[b][pl.ds(v * EMB_DIM + c * LANES, LANES)] = vals[c]

    def dstep(s, b, fire_next):
        bn = 1 - b
        j = j0 + s

        @pl.when(s >= 2)
        def _():
            drain_out(b)

        @pl.when(fire_next)
        def _():
            fire_in(j + 1, bn)

        drain_in(b)
        transpose_block(b)
        fire_out(j, b)

    fire_in(j0, 0)

    def dsuper(t, c2):
        s0 = 2 * t
        dstep(s0, 0, s0 + 1 < nj)
        dstep(s0 + 1, 1, s0 + 2 < nj)
        return c2

    lax.fori_loop(0, _VB_BASE // 2, dsuper, 0)

    @pl.when(nj > _VB_BASE)
    def _():
        dstep(_VB_BASE, 0, jnp.bool_(False))

    drain_out(0)
    drain_out(1)


@functools.partial(
    pl.kernel,
    mesh=_MESH,
    out_type=jax.ShapeDtypeStruct((VOCAB * EMB_DIM,), jnp.float32),
    scratch_types=[
        [pltpu.VMEM((8, 8, 128), jnp.float32) for _ in range(2)],
        [pltpu.VMEM((128 * EMB_DIM,), jnp.float32) for _ in range(2)],
        [pltpu.SemaphoreType.DMA for _ in range(2)],
        [pltpu.SemaphoreType.DMA for _ in range(2)],
    ],
    compiler_params=pltpu.CompilerParams(
        use_tc_tiling_on_sc=True, needs_layout_passes=False
    ),
)
def _detile_sc(tabt_hbm, t2_hbm, tbuf, sbuf, isem, osem):
    _detile_body(tabt_hbm, t2_hbm, tbuf, sbuf, isem, osem)


def _embed_body(x1d_hbm, table_hbm, out_hbm, idx_all, gbuf, stage, gsem, osem):
    wid = _worker_id()
    b0 = wid * BW

    pltpu.sync_copy(x1d_hbm.at[pl.ds(wid * BPW, BPW)], idx_all)

    def fire_gather(g, b):
        pltpu.async_copy(
            table_hbm.at[idx_all.at[pl.ds(g * G, G)]], gbuf[b], gsem[b]
        )

    def drain_gather(b):
        pltpu.make_async_copy(
            table_hbm.at[idx_all.at[pl.ds(0, G)]], gbuf[b], gsem[b]
        ).wait()

    def fire_store(g, b):
        h = g // NJ
        jj = g % NJ
        pltpu.async_copy(
            stage[b], out_hbm.at[h, :, pl.ds(b0 + jj * G, G)], osem[b]
        )

    def drain_store(b):
        pltpu.make_async_copy(
            stage[b], out_hbm.at[0, :, pl.ds(0, G)], osem[b]
        ).wait()

    iota16 = lax.broadcasted_iota(jnp.int32, (LANES,), 0)

    def transpose_scale(b):
        @plsc.parallel_loop(0, G // LANES, unroll=2)
        def _(m):
            rows16 = m * LANES + iota16
            for e0 in range(0, EMB_DIM, 16):
                vals = [
                    plsc.load_gather(
                        gbuf[b],
                        [rows16, jnp.full((LANES,), e0 + k, jnp.int32)],
                    )
                    * SCALE
                    for k in range(16)
                ]
                for k in range(16):
                    stage[b][e0 + k, pl.ds(m * LANES, LANES)] = vals[k]

    def halfstep(g, b, fire_next):
        bn = (b + 1) % NB

        @pl.when(g >= 2)
        def _():
            drain_store(bn)

        if fire_next:
            fire_gather(g + 1, bn)
        drain_gather(b)
        transpose_scale(b)
        fire_store(g, b)

    fire_gather(0, 0)

    nsup = (NBLK - 2) // NB

    def superstep(t, c2):
        for k in range(NB):
            halfstep(t * NB + k, k, fire_next=True)
        return c2

    lax.fori_loop(0, nsup, superstep, 0)

    halfstep(NBLK - 2, 0, fire_next=True)
    halfstep(NBLK - 1, 1, fire_next=False)
    drain_store(0)
    drain_store(1)


@functools.partial(
    pl.kernel,
    mesh=_MESH,
    out_type=jax.ShapeDtypeStruct((HIST, EMB_DIM, BATCH), jnp.float32),
    scratch_types=[
        pltpu.VMEM((BPW,), jnp.int32),
        [pltpu.VMEM((G, EMB_DIM), jnp.float32) for _ in range(NB)],
        [pltpu.VMEM((EMB_DIM, G), jnp.float32) for _ in range(NB)],
        [pltpu.SemaphoreType.DMA for _ in range(NB)],
        [pltpu.SemaphoreType.DMA for _ in range(NB)],
    ],
    compiler_params=pltpu.CompilerParams(
        use_tc_tiling_on_sc=False, needs_layout_passes=False
    ),
)
def _embed_sc(x1d_hbm, table_hbm, out_hbm, idx_all, gbuf, stage, gsem, osem):
    _embed_body(x1d_hbm, table_hbm, out_hbm, idx_all, gbuf, stage, gsem, osem)


def kernel(x, table):
    xt = x.T.astype(jnp.int32)
    x1d = _stage_idx_sc(xt)
    outp = _embed_sc(x1d, table)
    return outp.transpose(2, 0, 1)


# speedup vs baseline: 1.1369x; 1.1344x over previous
import functools
import math

import jax
import jax.numpy as jnp
from jax import lax
from jax.experimental import pallas as pl
from jax.experimental.pallas import tpu as pltpu
from jax.experimental.pallas import tpu_sc as plsc

VOCAB = 1000000
EMB_DIM = 64
BATCH = 16384
HIST = 50
NC = 2
NS = 16
NW = NC * NS
TOTAL = BATCH * HIST
BPW = TOTAL // NW
G = 128
NBLK = BPW // G
NB = 3
LANES = 16
SCALE = math.sqrt(EMB_DIM)
_MESH = plsc.VectorSubcoreMesh(core_axis_name="c", subcore_axis_name="s")


def _worker_id():
    return lax.axis_index("s") * NC + lax.axis_index("c")


def _embed_body(xf_hbm, table_hbm, out_hbm, idx_all, gbuf, gsem, osem):
    w = _worker_id()
    r0 = w * BPW

    pltpu.sync_copy(xf_hbm.at[pl.ds(r0, BPW)], idx_all)

    def fire_gather(g, b):
        pltpu.async_copy(
            table_hbm.at[idx_all.at[pl.ds(g * G, G)]], gbuf[b], gsem[b]
        )

    def drain_gather(b):
        pltpu.make_async_copy(
            table_hbm.at[idx_all.at[pl.ds(0, G)]], gbuf[b], gsem[b]
        ).wait()

    def fire_store(g, b):
        pltpu.async_copy(
            gbuf[b], out_hbm.at[pl.ds(r0 + g * G, G)], osem[b]
        )

    def drain_store(b):
        pltpu.make_async_copy(
            gbuf[b], out_hbm.at[pl.ds(0, G)], osem[b]
        ).wait()

    def scale_block(b):
        for v0 in range(0, G, 8):
            vals = [
                gbuf[b][v0 + r, pl.ds(c * LANES, LANES)] * SCALE
                for r in range(8)
                for c in range(EMB_DIM // LANES)
            ]
            i = 0
            for r in range(8):
                for c in range(EMB_DIM // LANES):
                    gbuf[b][v0 + r, pl.ds(c * LANES, LANES)] = vals[i]
                    i += 1

    def halfstep(g, b, fire_next):
        bn = (b + 1) % NB

        @pl.when(g >= 2)
        def _():
            drain_store(bn)

        if fire_next:
            fire_gather(g + 1, bn)
        drain_gather(b)
        scale_block(b)
        fire_store(g, b)

    fire_gather(0, 0)

    nsup = (NBLK - 2) // NB

    def superstep(t, c2):
        for k in range(NB):
            halfstep(t * NB + k, k, fire_next=True)
        return c2

    lax.fori_loop(0, nsup, superstep, 0)

    halfstep(NBLK - 2, 0, fire_next=True)
    halfstep(NBLK - 1, 1, fire_next=False)
    drain_store(0)
    drain_store(1)


@functools.partial(
    pl.kernel,
    mesh=_MESH,
    out_type=jax.ShapeDtypeStruct((TOTAL, EMB_DIM), jnp.float32),
    scratch_types=[
        pltpu.VMEM((BPW,), jnp.int32),
        [pltpu.VMEM((G, EMB_DIM), jnp.float32) for _ in range(NB)],
        [pltpu.SemaphoreType.DMA for _ in range(NB)],
        [pltpu.SemaphoreType.DMA for _ in range(NB)],
    ],
    compiler_params=pltpu.CompilerParams(
        use_tc_tiling_on_sc=False, needs_layout_passes=False
    ),
)
def _embed_sc(xf_hbm, table_hbm, out_hbm, idx_all, gbuf, gsem, osem):
    _embed_body(xf_hbm, table_hbm, out_hbm, idx_all, gbuf, gsem, osem)


def kernel(x, table):
    xf = x.reshape(TOTAL).astype(jnp.int32)
    out = _embed_sc(xf, table)
    return out.reshape(BATCH, HIST, EMB_DIM)


# speedup vs baseline: 1.1410x; 1.0036x over previous
import functools
import math

import jax
import jax.numpy as jnp
from jax import lax
from jax.experimental import pallas as pl
from jax.experimental.pallas import tpu as pltpu
from jax.experimental.pallas import tpu_sc as plsc

VOCAB = 1000000
EMB_DIM = 64
BATCH = 16384
HIST = 50
NC = 2
NS = 16
NW = NC * NS
TOTAL = BATCH * HIST
BPW = TOTAL // NW
G = 256
NBLK = BPW // G
NB = 3
LANES = 16
SCALE = math.sqrt(EMB_DIM)
_MESH = plsc.VectorSubcoreMesh(core_axis_name="c", subcore_axis_name="s")


def _worker_id():
    return lax.axis_index("s") * NC + lax.axis_index("c")


def _embed_body(xf_hbm, table_hbm, out_hbm, idx_all, gbuf, gsem, osem):
    w = _worker_id()
    r0 = w * BPW

    pltpu.sync_copy(xf_hbm.at[pl.ds(r0, BPW)], idx_all)

    def fire_gather(g, b):
        pltpu.async_copy(
            table_hbm.at[idx_all.at[pl.ds(g * G, G)]], gbuf[b], gsem[b]
        )

    def drain_gather(b):
        pltpu.make_async_copy(
            table_hbm.at[idx_all.at[pl.ds(0, G)]], gbuf[b], gsem[b]
        ).wait()

    def fire_store(g, b):
        pltpu.async_copy(
            gbuf[b], out_hbm.at[pl.ds(r0 + g * G, G)], osem[b]
        )

    def drain_store(b):
        pltpu.make_async_copy(
            gbuf[b], out_hbm.at[pl.ds(0, G)], osem[b]
        ).wait()

    def scale_block(b):
        for v0 in range(0, G, 8):
            vals = [
                gbuf[b][v0 + r, pl.ds(c * LANES, LANES)] * SCALE
                for r in range(8)
                for c in range(EMB_DIM // LANES)
            ]
            i = 0
            for r in range(8):
                for c in range(EMB_DIM // LANES):
                    gbuf[b][v0 + r, pl.ds(c * LANES, LANES)] = vals[i]
                    i += 1

    def halfstep(g, b, fire_next):
        bn = (b + 1) % NB

        @pl.when(g >= 2)
        def _():
            drain_store(bn)

        if fire_next:
            fire_gather(g + 1, bn)
        drain_gather(b)
        scale_block(b)
        fire_store(g, b)

    fire_gather(0, 0)

    nsup = (NBLK - 2) // NB

    def superstep(t, c2):
        for k in range(NB):
            halfstep(t * NB + k, k, fire_next=True)
        return c2

    lax.fori_loop(0, nsup, superstep, 0)

    for g in range(nsup * NB, NBLK):
        halfstep(g, g % NB, fire_next=(g + 1 < NBLK))
    drain_store((NBLK - 2) % NB)
    drain_store((NBLK - 1) % NB)


@functools.partial(
    pl.kernel,
    mesh=_MESH,
    out_type=jax.ShapeDtypeStruct((TOTAL, EMB_DIM), jnp.float32),
    scratch_types=[
        pltpu.VMEM((BPW,), jnp.int32),
        [pltpu.VMEM((G, EMB_DIM), jnp.float32) for _ in range(NB)],
        [pltpu.SemaphoreType.DMA for _ in range(NB)],
        [pltpu.SemaphoreType.DMA for _ in range(NB)],
    ],
    compiler_params=pltpu.CompilerParams(
        use_tc_tiling_on_sc=False, needs_layout_passes=False
    ),
)
def _embed_sc(xf_hbm, table_hbm, out_hbm, idx_all, gbuf, gsem, osem):
    _embed_body(xf_hbm, table_hbm, out_hbm, idx_all, gbuf, gsem, osem)


def kernel(x, table):
    xf = x.reshape(TOTAL).astype(jnp.int32)
    out = _embed_sc(xf, table)
    return out.reshape(BATCH, HIST, EMB_DIM)
